# baseline (speedup 1.0000x reference)
import functools

import jax
import jax.numpy as jnp
from jax.experimental import pallas as pl
from jax.experimental.pallas import tpu as pltpu

N = 10000
D = 128
RB_A = 400
RB_B = 400


def _deg_xs_kernel(adj_ref, x_ref, deg_ref, xs_ref):
    deg = jnp.sum(adj_ref[...], axis=1, keepdims=True) + 1.0
    deg_ref[...] = deg
    xs_ref[...] = (x_ref[...] * jax.lax.rsqrt(deg)).astype(jnp.bfloat16)


def _spmm_kernel(params_ref, adj_ref, xs_ref, deg_ref, x_ref, h0_ref, w_ref,
                 out_ref):
    a = adj_ref[...].astype(jnp.bfloat16)
    acc = jnp.dot(a, xs_ref[...], preferred_element_type=jnp.float32)
    theta = params_ref[0]
    alpha = params_ref[1]
    dinv_i = jax.lax.rsqrt(deg_ref[...])
    hi = dinv_i * acc + (dinv_i * dinv_i) * x_ref[...]
    support = (1.0 - alpha) * hi + alpha * h0_ref[...]
    out_ref[...] = (theta * jnp.dot(support, w_ref[...],
                                    preferred_element_type=jnp.float32)
                    + (1.0 - theta) * support)


@functools.partial(jax.jit, static_argnames=())
def _gcnii(x, adj, h0, w, theta, alpha):
    deg, xs = pl.pallas_call(
        _deg_xs_kernel,
        grid=(N // RB_A,),
        in_specs=[
            pl.BlockSpec((RB_A, N), lambda i: (i, 0)),
            pl.BlockSpec((RB_A, D), lambda i: (i, 0)),
        ],
        out_specs=[
            pl.BlockSpec((RB_A, 1), lambda i: (i, 0)),
            pl.BlockSpec((RB_A, D), lambda i: (i, 0)),
        ],
        out_shape=[
            jax.ShapeDtypeStruct((N, 1), jnp.float32),
            jax.ShapeDtypeStruct((N, D), jnp.bfloat16),
        ],
        compiler_params=pltpu.CompilerParams(
            dimension_semantics=("parallel",),
        ),
    )(adj, x)

    params = jnp.stack([theta, alpha]).astype(jnp.float32)
    out = pl.pallas_call(
        _spmm_kernel,
        grid=(N // RB_B,),
        in_specs=[
            pl.BlockSpec(memory_space=pltpu.SMEM),
            pl.BlockSpec((RB_B, N), lambda i: (i, 0)),
            pl.BlockSpec((N, D), lambda i: (0, 0)),
            pl.BlockSpec((RB_B, 1), lambda i: (i, 0)),
            pl.BlockSpec((RB_B, D), lambda i: (i, 0)),
            pl.BlockSpec((RB_B, D), lambda i: (i, 0)),
            pl.BlockSpec((D, D), lambda i: (0, 0)),
        ],
        out_specs=pl.BlockSpec((RB_B, D), lambda i: (i, 0)),
        out_shape=jax.ShapeDtypeStruct((N, D), jnp.float32),
        compiler_params=pltpu.CompilerParams(
            dimension_semantics=("parallel",),
        ),
    )(params, adj, xs, deg, x, h0, w)
    return out


def kernel(input, adj, h0, W, lamda, alpha, l):
    theta = jnp.log(jnp.asarray(lamda, dtype=jnp.float32)
                    / jnp.asarray(l, dtype=jnp.float32) + 1.0)
    alpha = jnp.asarray(alpha, dtype=jnp.float32)
    return _gcnii(input, adj, h0, W, theta, alpha)


# speedup vs baseline: 1.1296x; 1.1296x over previous
import functools

import jax
import jax.numpy as jnp
from jax.experimental import pallas as pl
from jax.experimental.pallas import tpu as pltpu

N = 10000
D = 128
RB_A = 400
RB_B = 400


def _deg_xs_kernel(adj_ref, x_ref, deg_ref, xs_ref, adj8_ref):
    a = adj_ref[...]
    deg = jnp.sum(a, axis=1, keepdims=True) + 1.0
    deg_ref[...] = deg
    xs_ref[...] = (x_ref[...] * jax.lax.rsqrt(deg)).astype(jnp.bfloat16)
    adj8_ref[...] = a.astype(jnp.float8_e4m3fn)


def _spmm_kernel(params_ref, adj_ref, xs_ref, deg_ref, x_ref, h0_ref, w_ref,
                 out_ref):
    a = adj_ref[...].astype(jnp.bfloat16)
    acc = jnp.dot(a, xs_ref[...], preferred_element_type=jnp.float32)
    theta = params_ref[0]
    alpha = params_ref[1]
    dinv_i = jax.lax.rsqrt(deg_ref[...])
    hi = dinv_i * acc + (dinv_i * dinv_i) * x_ref[...]
    support = (1.0 - alpha) * hi + alpha * h0_ref[...]
    out_ref[...] = (theta * jnp.dot(support, w_ref[...],
                                    preferred_element_type=jnp.float32)
                    + (1.0 - theta) * support)


@functools.partial(jax.jit, static_argnames=())
def _gcnii(x, adj, h0, w, theta, alpha):
    deg, xs, adj8 = pl.pallas_call(
        _deg_xs_kernel,
        grid=(N // RB_A,),
        in_specs=[
            pl.BlockSpec((RB_A, N), lambda i: (i, 0)),
            pl.BlockSpec((RB_A, D), lambda i: (i, 0)),
        ],
        out_specs=[
            pl.BlockSpec((RB_A, 1), lambda i: (i, 0)),
            pl.BlockSpec((RB_A, D), lambda i: (i, 0)),
            pl.BlockSpec((RB_A, N), lambda i: (i, 0)),
        ],
        out_shape=[
            jax.ShapeDtypeStruct((N, 1), jnp.float32),
            jax.ShapeDtypeStruct((N, D), jnp.bfloat16),
            jax.ShapeDtypeStruct((N, N), jnp.float8_e4m3fn),
        ],
        compiler_params=pltpu.CompilerParams(
            dimension_semantics=("parallel",),
        ),
    )(adj, x)

    params = jnp.stack([theta, alpha]).astype(jnp.float32)
    out = pl.pallas_call(
        _spmm_kernel,
        grid=(N // RB_B,),
        in_specs=[
            pl.BlockSpec(memory_space=pltpu.SMEM),
            pl.BlockSpec((RB_B, N), lambda i: (i, 0)),
            pl.BlockSpec((N, D), lambda i: (0, 0)),
            pl.BlockSpec((RB_B, 1), lambda i: (i, 0)),
            pl.BlockSpec((RB_B, D), lambda i: (i, 0)),
            pl.BlockSpec((RB_B, D), lambda i: (i, 0)),
            pl.BlockSpec((D, D), lambda i: (0, 0)),
        ],
        out_specs=pl.BlockSpec((RB_B, D), lambda i: (i, 0)),
        out_shape=jax.ShapeDtypeStruct((N, D), jnp.float32),
        compiler_params=pltpu.CompilerParams(
            dimension_semantics=("parallel",),
        ),
    )(params, adj8, xs, deg, x, h0, w)
    return out


def kernel(input, adj, h0, W, lamda, alpha, l):
    theta = jnp.log(jnp.asarray(lamda, dtype=jnp.float32)
                    / jnp.asarray(l, dtype=jnp.float32) + 1.0)
    alpha = jnp.asarray(alpha, dtype=jnp.float32)
    return _gcnii(input, adj, h0, W, theta, alpha)
